# baseline (speedup 1.0000x reference)
import jax
import jax.numpy as jnp
from jax.experimental import pallas as pl

_GROUP_ORDER = 16
_FACTOR = 2
_SUB = _GROUP_ORDER // _FACTOR
_F = 96


def _copy_body(in_ref, out_ref):
    out_ref[...] = in_ref[...]


def kernel(x):
    B, C, H, W = x.shape
    split = 4
    fs = _F // split
    return pl.pallas_call(
        _copy_body,
        grid=(B, _SUB, split),
        in_specs=[
            pl.BlockSpec((1, fs, H, W),
                         lambda b, g, j: (b, _FACTOR * split * g + j, 0, 0))
        ],
        out_specs=pl.BlockSpec((1, fs, H, W),
                               lambda b, g, j: (b, split * g + j, 0, 0)),
        out_shape=jax.ShapeDtypeStruct((B, _SUB * _F, H, W), jnp.float32),
    )(x)


# speedup vs baseline: 1.0920x; 1.0920x over previous
import functools

import jax
import jax.numpy as jnp
from jax import lax
from jax.experimental import pallas as pl
from jax.experimental.pallas import tpu as pltpu
from jax.experimental.pallas import tpu_sc as plsc

_GROUP_ORDER = 16
_FACTOR = 2
_SUB = _GROUP_ORDER // _FACTOR
_F = 96
_NC = 2
_NS = 16
_NW = _NC * _NS

_CCH = 4
_NBUF = 3


def kernel(x):
    B, C, H, W = x.shape
    nblocks = B * _SUB
    per_w = nblocks // _NW
    cpb = _F // _CCH
    nchunks = per_w * cpb

    mesh = plsc.VectorSubcoreMesh(core_axis_name="c", subcore_axis_name="s")

    @functools.partial(
        pl.kernel,
        mesh=mesh,
        out_type=jax.ShapeDtypeStruct((B, _SUB * _F, H, W), jnp.float32),
        scratch_types=(
            [pltpu.VMEM((_CCH, H, W), jnp.float32)] * _NBUF
            + [pltpu.SemaphoreType.DMA] * (2 * _NBUF)
        ),
    )
    def sc_copy(x_hbm, out_hbm, *scratch):
        bufs = scratch[:_NBUF]
        insems = scratch[_NBUF:2 * _NBUF]
        outsems = scratch[2 * _NBUF:3 * _NBUF]
        wid = lax.axis_index("s") * _NC + lax.axis_index("c")

        def src(i):
            j = wid * per_w + i // cpb
            b = j // _SUB
            g = j % _SUB
            c0 = (_FACTOR * _F) * g + _CCH * (i % cpb)
            return x_hbm.at[b, pl.ds(c0, _CCH)]

        def dst(i):
            j = wid * per_w + i // cpb
            b = j // _SUB
            g = j % _SUB
            c0 = _F * g + _CCH * (i % cpb)
            return out_hbm.at[b, pl.ds(c0, _CCH)]

        in_cp = [None] * nchunks
        out_cp = [None] * nchunks
        for t in range(nchunks + 1):
            if t < nchunks:
                b = t % _NBUF
                if t >= _NBUF:
                    out_cp[t - _NBUF].wait()
                in_cp[t] = pltpu.async_copy(src(t), bufs[b], insems[b])
            if t >= 1:
                i = t - 1
                b = i % _NBUF
                in_cp[i].wait()
                out_cp[i] = pltpu.async_copy(bufs[b], dst(i), outsems[b])
        for i in range(nchunks - _NBUF, nchunks):
            out_cp[i].wait()

    return sc_copy(x)


# speedup vs baseline: 1.1427x; 1.0464x over previous
import jax
import jax.numpy as jnp
from jax.experimental import pallas as pl

_GROUP_ORDER = 16
_FACTOR = 2
_SUB = _GROUP_ORDER // _FACTOR
_F = 96


def _copy_body(in_ref, out_ref):
    out_ref[...] = in_ref[...]


def kernel(x):
    B, C, H, W = x.shape
    return pl.pallas_call(
        _copy_body,
        grid=(B, _SUB),
        in_specs=[
            pl.BlockSpec((1, _F, H, W), lambda b, g: (b, _FACTOR * g, 0, 0))
        ],
        out_specs=pl.BlockSpec((1, _F, H, W), lambda b, g: (b, g, 0, 0)),
        out_shape=jax.ShapeDtypeStruct((B, _SUB * _F, H, W), jnp.float32),
    )(x)


# speedup vs baseline: 1.1486x; 1.0052x over previous
import jax
import jax.numpy as jnp
from jax.experimental import pallas as pl

_GROUP_ORDER = 16
_FACTOR = 2
_SUB = _GROUP_ORDER // _FACTOR
_F = 96


def _copy_body(in_ref, out_ref):
    out_ref[...] = in_ref[...]


def kernel(x):
    B, C, H, W = x.shape
    return pl.pallas_call(
        _copy_body,
        grid=(B // 2, _SUB),
        in_specs=[
            pl.BlockSpec((2, _F, H, W), lambda b, g: (b, _FACTOR * g, 0, 0))
        ],
        out_specs=pl.BlockSpec((2, _F, H, W), lambda b, g: (b, g, 0, 0)),
        out_shape=jax.ShapeDtypeStruct((B, _SUB * _F, H, W), jnp.float32),
    )(x)
